# baseline (speedup 1.0000x reference)
import jax
import jax.numpy as jnp
from jax import lax
from jax.experimental import pallas as pl
from jax.experimental.pallas import tpu as pltpu
from jax.experimental.pallas import tpu_sc as plsc

NC = 2
NS = 16
L = 16

CHUNK = 128
GROUPS = CHUNK // L
ACC = 2

B = 32768
E = 64


def _merge(a, b):
    am1, ai1, am2, ai2 = a
    bm1, bi1, bm2, bi2 = b
    atake = (am1 > bm1) | ((am1 == bm1) & (ai1 < bi1))
    m1 = jnp.where(atake, am1, bm1)
    i1 = jnp.where(atake, ai1, bi1)
    x = jnp.where(atake, bm1, am1)
    xi = jnp.where(atake, bi1, ai1)
    ytake = (am2 > bm2) | ((am2 == bm2) & (ai2 < bi2))
    y = jnp.where(ytake, am2, bm2)
    yi = jnp.where(ytake, ai2, bi2)
    xtake = (x > y) | ((x == y) & (xi < yi))
    m2 = jnp.where(xtake, x, y)
    i2 = jnp.where(xtake, xi, yi)
    return m1, i1, m2, i2


def _router_body(logits_hbm, out_hbm, raw0, raw1, out0, out1, idx_v,
                 sin0, sin1, sout0, sout1):
    rows_per_worker = B // (NC * NS)
    chunks = rows_per_worker // CHUNK

    raws = [raw0, raw1]
    outs = [out0, out1]
    sins = [sin0, sin1]
    souts = [sout0, sout1]

    wid = lax.axis_index("s") * NC + lax.axis_index("c")
    wbase = wid * rows_per_worker

    iota = lax.iota(jnp.int32, L)
    zeros = jnp.zeros((L,), jnp.float32)
    neg_inf = jnp.full((L,), -jnp.inf, jnp.float32)
    izero = jnp.zeros((L,), jnp.int32)
    ione = jnp.full((L,), 1, jnp.int32)

    cols_per_acc = E // ACC

    def _rbase(chunk):
        return wbase + chunk * CHUNK

    dma_in = {}
    dma_out = {}
    dma_in[0] = pltpu.async_copy(
        logits_hbm.at[pl.ds(_rbase(0), CHUNK)], raws[0], sins[0])

    def _zero_row(r, c):
        for q in range(E // L):
            out0[r, pl.ds(q * L, L)] = zeros
            out1[r, pl.ds(q * L, L)] = zeros
        return c

    lax.fori_loop(0, CHUNK, _zero_row, 0)

    def _make_group_body(raw_v, out_v, buf):
        def _group_body(g):
            row = g * L + iota
            m1 = [neg_inf] * ACC
            m2 = [neg_inf] * ACC
            i1 = [izero] * ACC
            i2 = [izero] * ACC
            cidx = [(iota + (q * cols_per_acc)) & (E - 1) for q in range(ACC)]
            for _t in range(cols_per_acc):
                for q in range(ACC):
                    v = plsc.load_gather(raw_v, [row, cidx[q]])
                    gt1 = v > m1[q]
                    gt2 = v > m2[q]
                    m2[q] = jnp.where(gt1, m1[q], jnp.where(gt2, v, m2[q]))
                    i2[q] = jnp.where(gt1, i1[q], jnp.where(gt2, cidx[q], i2[q]))
                    m1[q] = jnp.where(gt1, v, m1[q])
                    i1[q] = jnp.where(gt1, cidx[q], i1[q])
                    cidx[q] = (cidx[q] + ione) & (E - 1)
            states = [(m1[q], i1[q], m2[q], i2[q]) for q in range(ACC)]
            while len(states) > 1:
                states = [_merge(states[2 * k], states[2 * k + 1])
                          for k in range(len(states) // 2)]
            tm1, ti1, tm2, ti2 = states[0]

            e = jnp.exp(tm2 - tm1)
            s = 1.0 + e
            w1 = 1.0 / s
            w2 = e / s
            plsc.store_scatter(out_v, [row, ti1], w1)
            plsc.store_scatter(out_v, [row, ti2], w2)
            base = (buf * GROUPS + g) * 2 * L
            idx_v[pl.ds(base, L)] = ti1
            idx_v[pl.ds(base + L, L)] = ti2
        return _group_body

    def _make_rezero_body(out_v, buf):
        def _rezero_body(g):
            row = g * L + iota
            base = (buf * GROUPS + g) * 2 * L
            c1 = idx_v[pl.ds(base, L)]
            c2 = idx_v[pl.ds(base + L, L)]
            plsc.store_scatter(out_v, [row, c1], zeros)
            plsc.store_scatter(out_v, [row, c2], zeros)
        return _rezero_body

    for chunk in range(chunks):
        cur = chunk % 2
        if chunk + 1 < chunks:
            dma_in[chunk + 1] = pltpu.async_copy(
                logits_hbm.at[pl.ds(_rbase(chunk + 1), CHUNK)],
                raws[1 - cur], sins[1 - cur])
        dma_in[chunk].wait()
        if chunk >= 2:
            dma_out[chunk - 2].wait()
            plsc.parallel_loop(0, GROUPS)(_make_rezero_body(outs[cur], cur))
        plsc.parallel_loop(0, GROUPS)(_make_group_body(raws[cur], outs[cur], cur))
        dma_out[chunk] = pltpu.async_copy(
            outs[cur], out_hbm.at[pl.ds(_rbase(chunk), CHUNK)], souts[cur])
    dma_out[chunks - 2].wait()
    dma_out[chunks - 1].wait()


@jax.jit
def _router(logits):
    k = pl.kernel(
        _router_body,
        out_type=jax.ShapeDtypeStruct((B, E), jnp.float32),
        mesh=plsc.VectorSubcoreMesh(
            core_axis_name="c", subcore_axis_name="s",
            num_cores=NC, num_subcores=NS,
        ),
        scratch_types=[
            pltpu.VMEM((CHUNK, E), jnp.float32),
            pltpu.VMEM((CHUNK, E), jnp.float32),
            pltpu.VMEM((CHUNK, E), jnp.float32),
            pltpu.VMEM((CHUNK, E), jnp.float32),
            pltpu.VMEM((2 * GROUPS * 2 * L,), jnp.int32),
            pltpu.SemaphoreType.DMA,
            pltpu.SemaphoreType.DMA,
            pltpu.SemaphoreType.DMA,
            pltpu.SemaphoreType.DMA,
        ],
        compiler_params=pltpu.CompilerParams(
            needs_layout_passes=False,
            disable_bounds_checks=True,
            disable_semaphore_checks=True,
        ),
    )
    return k(logits)


def kernel(logits):
    return _router(logits)


# speedup vs baseline: 1.2419x; 1.2419x over previous
import jax
import jax.numpy as jnp
from jax import lax
from jax.experimental import pallas as pl
from jax.experimental.pallas import tpu as pltpu
from jax.experimental.pallas import tpu_sc as plsc

NC = 2
NS = 16
L = 16

CHUNK = 128
GROUPS = CHUNK // L
ACC = 2

B = 32768
E = 64


def _merge(a, b):
    am1, ai1, am2, ai2 = a
    bm1, bi1, bm2, bi2 = b
    atake = (am1 > bm1) | ((am1 == bm1) & (ai1 < bi1))
    m1 = jnp.where(atake, am1, bm1)
    i1 = jnp.where(atake, ai1, bi1)
    x = jnp.where(atake, bm1, am1)
    xi = jnp.where(atake, bi1, ai1)
    ytake = (am2 > bm2) | ((am2 == bm2) & (ai2 < bi2))
    y = jnp.where(ytake, am2, bm2)
    yi = jnp.where(ytake, ai2, bi2)
    xtake = (x > y) | ((x == y) & (xi < yi))
    m2 = jnp.where(xtake, x, y)
    i2 = jnp.where(xtake, xi, yi)
    return m1, i1, m2, i2


def _router_body(logits_hbm, out_hbm, raw0, raw1, out0, out1, idx_v,
                 sin0, sin1, sout0, sout1):
    rows_per_worker = B // (NC * NS)
    chunks = rows_per_worker // CHUNK

    raws = [raw0, raw1]
    outs = [out0, out1]
    sins = [sin0, sin1]
    souts = [sout0, sout1]

    wid = lax.axis_index("s") * NC + lax.axis_index("c")
    wbase = wid * rows_per_worker

    iota = lax.iota(jnp.int32, L)
    zeros = jnp.zeros((L,), jnp.float32)
    neg_inf = jnp.full((L,), -jnp.inf, jnp.float32)
    izero = jnp.zeros((L,), jnp.int32)
    ione = jnp.full((L,), 1, jnp.int32)

    cols_per_acc = E // ACC

    def _rbase(chunk):
        return wbase + chunk * CHUNK

    dma_in = {}
    dma_out = {}
    dma_in[0] = pltpu.async_copy(
        logits_hbm.at[pl.ds(_rbase(0), CHUNK)], raws[0], sins[0])

    def _zero_row(r, c):
        for q in range(E // L):
            out0[r, pl.ds(q * L, L)] = zeros
            out1[r, pl.ds(q * L, L)] = zeros
        return c

    lax.fori_loop(0, CHUNK, _zero_row, 0)

    def _make_group_body(raw_v, out_v, buf):
        def _group_body(g, c):
            row = g * L + iota
            m1 = [neg_inf] * ACC
            m2 = [neg_inf] * ACC
            i1 = [izero] * ACC
            i2 = [izero] * ACC
            cidx = [(iota + (q * cols_per_acc)) & (E - 1) for q in range(ACC)]
            for _t in range(cols_per_acc):
                for q in range(ACC):
                    v = plsc.load_gather(raw_v, [row, cidx[q]])
                    gt1 = v > m1[q]
                    gt2 = v > m2[q]
                    m2[q] = jnp.where(gt1, m1[q], jnp.where(gt2, v, m2[q]))
                    i2[q] = jnp.where(gt1, i1[q], jnp.where(gt2, cidx[q], i2[q]))
                    m1[q] = jnp.where(gt1, v, m1[q])
                    i1[q] = jnp.where(gt1, cidx[q], i1[q])
                    cidx[q] = (cidx[q] + ione) & (E - 1)
            states = [(m1[q], i1[q], m2[q], i2[q]) for q in range(ACC)]
            while len(states) > 1:
                states = [_merge(states[2 * k], states[2 * k + 1])
                          for k in range(len(states) // 2)]
            tm1, ti1, tm2, ti2 = states[0]

            e = jnp.exp(tm2 - tm1)
            s = 1.0 + e
            w1 = 1.0 / s
            w2 = e / s
            plsc.store_scatter(out_v, [row, ti1], w1)
            plsc.store_scatter(out_v, [row, ti2], w2)
            base = (buf * GROUPS + g) * 2 * L
            idx_v[pl.ds(base, L)] = ti1
            idx_v[pl.ds(base + L, L)] = ti2
            return c
        return _group_body

    def _make_rezero_body(out_v, buf):
        def _rezero_body(g, c):
            row = g * L + iota
            base = (buf * GROUPS + g) * 2 * L
            c1 = idx_v[pl.ds(base, L)]
            c2 = idx_v[pl.ds(base + L, L)]
            plsc.store_scatter(out_v, [row, c1], zeros)
            plsc.store_scatter(out_v, [row, c2], zeros)
            return c
        return _rezero_body

    for chunk in range(chunks):
        cur = chunk % 2
        if chunk + 1 < chunks:
            dma_in[chunk + 1] = pltpu.async_copy(
                logits_hbm.at[pl.ds(_rbase(chunk + 1), CHUNK)],
                raws[1 - cur], sins[1 - cur])
        dma_in[chunk].wait()
        if chunk >= 2:
            dma_out[chunk - 2].wait()
            lax.fori_loop(0, GROUPS, _make_rezero_body(outs[cur], cur), 0)
        lax.fori_loop(0, GROUPS, _make_group_body(raws[cur], outs[cur], cur), 0)
        dma_out[chunk] = pltpu.async_copy(
            outs[cur], out_hbm.at[pl.ds(_rbase(chunk), CHUNK)], souts[cur])
    dma_out[chunks - 2].wait()
    dma_out[chunks - 1].wait()


@jax.jit
def _router(logits):
    k = pl.kernel(
        _router_body,
        out_type=jax.ShapeDtypeStruct((B, E), jnp.float32),
        mesh=plsc.VectorSubcoreMesh(
            core_axis_name="c", subcore_axis_name="s",
            num_cores=NC, num_subcores=NS,
        ),
        scratch_types=[
            pltpu.VMEM((CHUNK, E), jnp.float32),
            pltpu.VMEM((CHUNK, E), jnp.float32),
            pltpu.VMEM((CHUNK, E), jnp.float32),
            pltpu.VMEM((CHUNK, E), jnp.float32),
            pltpu.VMEM((2 * GROUPS * 2 * L,), jnp.int32),
            pltpu.SemaphoreType.DMA,
            pltpu.SemaphoreType.DMA,
            pltpu.SemaphoreType.DMA,
            pltpu.SemaphoreType.DMA,
        ],
        compiler_params=pltpu.CompilerParams(
            needs_layout_passes=False,
            disable_bounds_checks=True,
            disable_semaphore_checks=True,
        ),
    )
    return k(logits)


def kernel(logits):
    return _router(logits)
